# baseline (speedup 1.0000x reference)
import dataclasses
import functools

import jax
import jax.numpy as jnp
from jax import lax
from jax.experimental import pallas as pl
from jax.experimental.pallas import tpu as pltpu
from jax.experimental.pallas import tpu_sc as plsc

_N = 10000
_E = 320000
_D = 128
_R = 8
_B = 4

_NC = 2
_NS = 16
_NW = _NC * _NS
_CH = 128
_CPT = 80
_EPT = _CH * _CPT
_EPAD = _NW * _EPT
_CPH = _CPT // 2
_RPT = 624

_BN = 1000


def _xw_body(coeff_ref, bases_ref, x_ref, out_ref):
    w = coeff_ref[0, 0, 0] * bases_ref[0]
    for b in range(1, _B):
        w = w + coeff_ref[0, 0, b] * bases_ref[b]
    out_ref[0] = lax.dot_general(
        x_ref[...], w, (((1,), (0,)), ((), ())),
        preferred_element_type=jnp.float32)


_xw_call = pl.pallas_call(
    _xw_body,
    grid=(_R, _N // _BN),
    in_specs=[
        pl.BlockSpec((1, 1, _B), lambda r, i: (r, 0, 0)),
        pl.BlockSpec((_B, _D, _D), lambda r, i: (0, 0, 0)),
        pl.BlockSpec((_BN, _D), lambda r, i: (i, 0)),
    ],
    out_specs=pl.BlockSpec((1, _BN, _D), lambda r, i: (r, i, 0)),
    out_shape=jax.ShapeDtypeStruct((_R, _N, _D), jnp.float32),
)


_sc_mesh = plsc.VectorSubcoreMesh(core_axis_name="c", subcore_axis_name="s")

_sc_params = pltpu.CompilerParams()
if "needs_layout_passes" in pltpu.CompilerParams.__dataclass_fields__:
    _sc_params = dataclasses.replace(_sc_params, needs_layout_passes=False)


@functools.partial(
    pl.kernel,
    out_type=jax.ShapeDtypeStruct((_NC, _N, _D), jnp.float32),
    mesh=_sc_mesh,
    compiler_params=_sc_params,
    scratch_types=[
        pltpu.VMEM((_CPH, _CH), jnp.int32),
        pltpu.VMEM((_CPH, _CH), jnp.int32),
        pltpu.VMEM((_CPH, _CH), jnp.float32),
        pltpu.VMEM((_CH, _D), jnp.float32),
        pltpu.VMEM((_CH, _D), jnp.float32),
        pltpu.SemaphoreType.DMA,
        pltpu.SemaphoreType.DMA,
        pltpu.VMEM_SHARED((_N, _D), jnp.float32),
    ],
)
def _sc_scatter(table_hbm, gidx_hbm, dst_hbm, norm_hbm, zeros_hbm, out_hbm,
                gidx_v, dst_v, norm_v, rows_a, rows_b, sem_a, sem_b, acc_sh):
    c = lax.axis_index("c")
    s = lax.axis_index("s")
    wid = c * _NS + s
    pltpu.sync_copy(zeros_hbm, acc_sh.at[pl.ds(s * _RPT, _RPT)])

    @pl.when(s == _NS - 1)
    def _():
        pltpu.sync_copy(zeros_hbm.at[pl.ds(0, _N - _NS * _RPT)],
                        acc_sh.at[pl.ds(_NS * _RPT, _N - _NS * _RPT)])

    plsc.subcore_barrier()

    def _scale(i, rows):
        @pl.loop(0, _CH, step=8)
        def _edge(e0):
            for k in range(8):
                nv = plsc.load_gather(
                    norm_v, [jnp.full((16,), i, jnp.int32),
                             jnp.full((16,), e0 + k, jnp.int32)])
                for j in range(_D // 16):
                    sl = (e0 + k, pl.ds(j * 16, 16))
                    rows[sl] = rows[sl] * nv

    def _gather_start(i, rows, sem):
        pltpu.async_copy(table_hbm.at[gidx_v.at[i]], rows, sem)

    def _gather_wait(rows, sem):
        pltpu.make_async_copy(table_hbm.at[pl.ds(0, _CH)], rows, sem).wait()

    def _process(i, rows):
        _scale(i, rows)
        pltpu.sync_copy(rows, acc_sh.at[dst_v.at[i]], add=True)

    for h in range(_CPT // _CPH):
        pltpu.sync_copy(gidx_hbm.at[wid, pl.ds(h * _CPH, _CPH)], gidx_v)
        pltpu.sync_copy(dst_hbm.at[wid, pl.ds(h * _CPH, _CPH)], dst_v)
        pltpu.sync_copy(norm_hbm.at[wid, pl.ds(h * _CPH, _CPH)], norm_v)
        _gather_start(0, rows_a, sem_a)

        @pl.loop(0, _CPH - 2, step=2)
        def _chunk(i):
            _gather_wait(rows_a, sem_a)
            _gather_start(i + 1, rows_b, sem_b)
            _process(i, rows_a)
            _gather_wait(rows_b, sem_b)
            _gather_start(i + 2, rows_a, sem_a)
            _process(i + 1, rows_b)

        _gather_wait(rows_a, sem_a)
        _gather_start(_CPH - 1, rows_b, sem_b)
        _process(_CPH - 2, rows_a)
        _gather_wait(rows_b, sem_b)
        _process(_CPH - 1, rows_b)

    plsc.subcore_barrier()
    pltpu.sync_copy(acc_sh.at[pl.ds(s * _RPT, _RPT)],
                    out_hbm.at[c, pl.ds(s * _RPT, _RPT)])

    @pl.when(s == _NS - 1)
    def _():
        pltpu.sync_copy(acc_sh.at[pl.ds(_NS * _RPT, _N - _NS * _RPT)],
                        out_hbm.at[c, pl.ds(_NS * _RPT, _N - _NS * _RPT)])


def _fin_body(x_ref, lw_ref, bias_ref, agg_ref, out_ref):
    sl = lax.dot_general(
        x_ref[...], lw_ref[...], (((1,), (0,)), ((), ())),
        preferred_element_type=jnp.float32)
    out_ref[...] = jnp.maximum(
        sl + agg_ref[0] + agg_ref[1] + bias_ref[...], 0.0)


_fin_call = pl.pallas_call(
    _fin_body,
    grid=(_N // _BN,),
    in_specs=[
        pl.BlockSpec((_BN, _D), lambda i: (i, 0)),
        pl.BlockSpec((_D, _D), lambda i: (0, 0)),
        pl.BlockSpec((1, _D), lambda i: (0, 0)),
        pl.BlockSpec((_NC, _BN, _D), lambda i: (0, i, 0)),
    ],
    out_specs=pl.BlockSpec((_BN, _D), lambda i: (i, 0)),
    out_shape=jax.ShapeDtypeStruct((_N, _D), jnp.float32),
)


def kernel(x, edge_index, etype, norm, bases, coeff, loop_weight, bias):
    src = edge_index[0]
    dst = edge_index[1]
    gidx = etype.astype(jnp.int32) * _N + src.astype(jnp.int32)
    pad = _EPAD - _E
    gidx_p = jnp.concatenate(
        [gidx, jnp.zeros((pad,), jnp.int32)]).reshape(_NW, _CPT, _CH)
    dst_p = jnp.concatenate(
        [dst.astype(jnp.int32), jnp.zeros((pad,), jnp.int32)]
    ).reshape(_NW, _CPT, _CH)
    norm_p = jnp.concatenate(
        [norm[:, 0].astype(jnp.float32), jnp.zeros((pad,), jnp.float32)]
    ).reshape(_NW, _CPT, _CH)

    xw = _xw_call(coeff.reshape(_R, 1, _B), bases, x)
    table = xw.reshape(_R * _N, _D)
    zeros = jnp.zeros((_RPT, _D), jnp.float32)
    agg = _sc_scatter(table, gidx_p, dst_p, norm_p, zeros)
    return _fin_call(x, loop_weight, bias.reshape(1, _D), agg)


# speedup vs baseline: 1.0860x; 1.0860x over previous
import dataclasses
import functools

import jax
import jax.numpy as jnp
from jax import lax
from jax.experimental import pallas as pl
from jax.experimental.pallas import tpu as pltpu
from jax.experimental.pallas import tpu_sc as plsc

_N = 10000
_E = 320000
_D = 128
_R = 8
_B = 4

_NC = 2
_NS = 16
_NW = _NC * _NS
_CH = 128
_CPT = 80
_EPT = _CH * _CPT
_EPAD = _NW * _EPT
_CPH = _CPT // 2
_RPT = 624

_BN = 1000


def _xw_body(coeff_ref, bases_lo_ref, bases_hi_ref, x_ref, out_ref):
    w_lo = coeff_ref[0, 0, 0] * bases_lo_ref[0]
    w_hi = coeff_ref[0, 0, 0] * bases_hi_ref[0]
    for b in range(1, _B):
        w_lo = w_lo + coeff_ref[0, 0, b] * bases_lo_ref[b]
        w_hi = w_hi + coeff_ref[0, 0, b] * bases_hi_ref[b]
    x = x_ref[...]
    dn = (((1,), (0,)), ((), ()))
    y_lo = lax.dot_general(x, w_lo, dn, preferred_element_type=jnp.float32)
    y_hi = lax.dot_general(x, w_hi, dn, preferred_element_type=jnp.float32)
    lo16 = lax.bitcast_convert_type(
        y_lo.astype(jnp.bfloat16), jnp.uint16).astype(jnp.int32)
    hi16 = lax.bitcast_convert_type(
        y_hi.astype(jnp.bfloat16), jnp.uint16).astype(jnp.int32)
    out_ref[0] = jnp.bitwise_or(lax.shift_left(hi16, 16), lo16)


_DW = _D // 2

_xw_call = pl.pallas_call(
    _xw_body,
    grid=(_R, _N // _BN),
    in_specs=[
        pl.BlockSpec((1, 1, _B), lambda r, i: (r, 0, 0)),
        pl.BlockSpec((_B, _D, _DW), lambda r, i: (0, 0, 0)),
        pl.BlockSpec((_B, _D, _DW), lambda r, i: (0, 0, 0)),
        pl.BlockSpec((_BN, _D), lambda r, i: (i, 0)),
    ],
    out_specs=pl.BlockSpec((1, _BN, _DW), lambda r, i: (r, i, 0)),
    out_shape=jax.ShapeDtypeStruct((_R, _N, _DW), jnp.int32),
)

_COLS_LO = [32 * (t // 16) + (t % 16) for t in range(_DW)]
_COLS_HI = [c + 16 for c in _COLS_LO]


_sc_mesh = plsc.VectorSubcoreMesh(core_axis_name="c", subcore_axis_name="s")

_sc_params = pltpu.CompilerParams(
    needs_layout_passes=False, use_tc_tiling_on_sc=False)


@functools.partial(
    pl.kernel,
    out_type=jax.ShapeDtypeStruct((_NC, _N, _D), jnp.float32),
    mesh=_sc_mesh,
    compiler_params=_sc_params,
    scratch_types=[
        pltpu.VMEM((_CPH, _CH), jnp.int32),
        pltpu.VMEM((_CPH, _CH), jnp.int32),
        pltpu.VMEM((_CPH, _CH), jnp.float32),
        pltpu.VMEM((_CH, _DW), jnp.int32),
        pltpu.VMEM((_CH, _DW), jnp.int32),
        pltpu.VMEM((_CH, _D), jnp.float32),
        pltpu.SemaphoreType.DMA,
        pltpu.SemaphoreType.DMA,
        pltpu.VMEM_SHARED((_N, _D), jnp.float32),
    ],
)
def _sc_scatter(table_hbm, gidx_hbm, dst_hbm, norm_hbm, zeros_hbm, out_hbm,
                gidx_v, dst_v, norm_v, rows_a, rows_b, rows_f, sem_a, sem_b,
                acc_sh):
    c = lax.axis_index("c")
    s = lax.axis_index("s")
    wid = c * _NS + s
    pltpu.sync_copy(zeros_hbm, acc_sh.at[pl.ds(s * _RPT, _RPT)])

    @pl.when(s == _NS - 1)
    def _():
        pltpu.sync_copy(zeros_hbm.at[pl.ds(0, _N - _NS * _RPT)],
                        acc_sh.at[pl.ds(_NS * _RPT, _N - _NS * _RPT)])

    plsc.subcore_barrier()

    def _scale(i, rows):
        @pl.loop(0, _CH, step=4)
        def _edge(e0):
            for k in range(4):
                e = e0 + k
                nv = plsc.load_gather(
                    norm_v, [jnp.full((16,), i, jnp.int32),
                             jnp.full((16,), e, jnp.int32)])
                for q in range(_D // 32):
                    w = rows[e, pl.ds(q * 16, 16)]
                    lo = plsc.bitcast(lax.shift_left(w, 16), jnp.float32)
                    hi = plsc.bitcast(
                        jnp.bitwise_and(w, jnp.int32(-65536)), jnp.float32)
                    rows_f[e, pl.ds(q * 32, 16)] = lo * nv
                    rows_f[e, pl.ds(q * 32 + 16, 16)] = hi * nv

    def _gather_start(i, rows, sem):
        pltpu.async_copy(table_hbm.at[gidx_v.at[i]], rows, sem)

    def _gather_wait(rows, sem):
        pltpu.make_async_copy(table_hbm.at[pl.ds(0, _CH)], rows, sem).wait()

    def _process(i, rows):
        _scale(i, rows)
        pltpu.sync_copy(rows_f, acc_sh.at[dst_v.at[i]], add=True)

    for h in range(_CPT // _CPH):
        pltpu.sync_copy(gidx_hbm.at[wid, pl.ds(h * _CPH, _CPH)], gidx_v)
        pltpu.sync_copy(dst_hbm.at[wid, pl.ds(h * _CPH, _CPH)], dst_v)
        pltpu.sync_copy(norm_hbm.at[wid, pl.ds(h * _CPH, _CPH)], norm_v)
        _gather_start(0, rows_a, sem_a)

        @pl.loop(0, _CPH - 2, step=2)
        def _chunk(i):
            _gather_wait(rows_a, sem_a)
            _gather_start(i + 1, rows_b, sem_b)
            _process(i, rows_a)
            _gather_wait(rows_b, sem_b)
            _gather_start(i + 2, rows_a, sem_a)
            _process(i + 1, rows_b)

        _gather_wait(rows_a, sem_a)
        _gather_start(_CPH - 1, rows_b, sem_b)
        _process(_CPH - 2, rows_a)
        _gather_wait(rows_b, sem_b)
        _process(_CPH - 1, rows_b)

    plsc.subcore_barrier()
    pltpu.sync_copy(acc_sh.at[pl.ds(s * _RPT, _RPT)],
                    out_hbm.at[c, pl.ds(s * _RPT, _RPT)])

    @pl.when(s == _NS - 1)
    def _():
        pltpu.sync_copy(acc_sh.at[pl.ds(_NS * _RPT, _N - _NS * _RPT)],
                        out_hbm.at[c, pl.ds(_NS * _RPT, _N - _NS * _RPT)])


def _fin_body(x_ref, lw_ref, bias_ref, agg_ref, out_ref):
    sl = lax.dot_general(
        x_ref[...], lw_ref[...], (((1,), (0,)), ((), ())),
        preferred_element_type=jnp.float32)
    out_ref[...] = jnp.maximum(
        sl + agg_ref[0] + agg_ref[1] + bias_ref[...], 0.0)


_fin_call = pl.pallas_call(
    _fin_body,
    grid=(_N // _BN,),
    in_specs=[
        pl.BlockSpec((_BN, _D), lambda i: (i, 0)),
        pl.BlockSpec((_D, _D), lambda i: (0, 0)),
        pl.BlockSpec((1, _D), lambda i: (0, 0)),
        pl.BlockSpec((_NC, _BN, _D), lambda i: (0, i, 0)),
    ],
    out_specs=pl.BlockSpec((_BN, _D), lambda i: (i, 0)),
    out_shape=jax.ShapeDtypeStruct((_N, _D), jnp.float32),
)


def kernel(x, edge_index, etype, norm, bases, coeff, loop_weight, bias):
    src = edge_index[0]
    dst = edge_index[1]
    gidx = etype.astype(jnp.int32) * _N + src.astype(jnp.int32)
    pad = _EPAD - _E
    gidx_p = jnp.concatenate(
        [gidx, jnp.zeros((pad,), jnp.int32)]).reshape(_NW, _CPT, _CH)
    dst_p = jnp.concatenate(
        [dst.astype(jnp.int32), jnp.zeros((pad,), jnp.int32)]
    ).reshape(_NW, _CPT, _CH)
    norm_p = jnp.concatenate(
        [norm[:, 0].astype(jnp.float32), jnp.zeros((pad,), jnp.float32)]
    ).reshape(_NW, _CPT, _CH)

    bases_lo = bases[:, :, jnp.array(_COLS_LO, dtype=jnp.int32)]
    bases_hi = bases[:, :, jnp.array(_COLS_HI, dtype=jnp.int32)]
    xw = _xw_call(coeff.reshape(_R, 1, _B), bases_lo, bases_hi, x)
    table = xw.reshape(_R * _N, _DW)
    zeros = jnp.zeros((_RPT, _D), jnp.float32)
    agg = _sc_scatter(table, gidx_p, dst_p, norm_p, zeros)
    return _fin_call(x, loop_weight, bias.reshape(1, _D), agg)


# speedup vs baseline: 1.0974x; 1.0105x over previous
import dataclasses
import functools

import jax
import jax.numpy as jnp
from jax import lax
from jax.experimental import pallas as pl
from jax.experimental.pallas import tpu as pltpu
from jax.experimental.pallas import tpu_sc as plsc

_N = 10000
_E = 320000
_D = 128
_R = 8
_B = 4

_NC = 2
_NS = 16
_NW = _NC * _NS
_CH = 128
_CPT = 80
_EPT = _CH * _CPT
_EPAD = _NW * _EPT
_CPH = _CPT // 2
_RPT = 624

_BN = 1000


def _xw_body(coeff_ref, bases_lo_ref, bases_hi_ref, x_ref, out_ref):
    w_lo = coeff_ref[0, 0, 0] * bases_lo_ref[0]
    w_hi = coeff_ref[0, 0, 0] * bases_hi_ref[0]
    for b in range(1, _B):
        w_lo = w_lo + coeff_ref[0, 0, b] * bases_lo_ref[b]
        w_hi = w_hi + coeff_ref[0, 0, b] * bases_hi_ref[b]
    x = x_ref[...]
    dn = (((1,), (0,)), ((), ()))
    y_lo = lax.dot_general(x, w_lo, dn, preferred_element_type=jnp.float32)
    y_hi = lax.dot_general(x, w_hi, dn, preferred_element_type=jnp.float32)
    lo16 = lax.bitcast_convert_type(
        y_lo.astype(jnp.bfloat16), jnp.uint16).astype(jnp.int32)
    hi16 = lax.bitcast_convert_type(
        y_hi.astype(jnp.bfloat16), jnp.uint16).astype(jnp.int32)
    out_ref[0] = jnp.bitwise_or(lax.shift_left(hi16, 16), lo16)


_DW = _D // 2

_xw_call = pl.pallas_call(
    _xw_body,
    grid=(_R, _N // _BN),
    in_specs=[
        pl.BlockSpec((1, 1, _B), lambda r, i: (r, 0, 0)),
        pl.BlockSpec((_B, _D, _DW), lambda r, i: (0, 0, 0)),
        pl.BlockSpec((_B, _D, _DW), lambda r, i: (0, 0, 0)),
        pl.BlockSpec((_BN, _D), lambda r, i: (i, 0)),
    ],
    out_specs=pl.BlockSpec((1, _BN, _DW), lambda r, i: (r, i, 0)),
    out_shape=jax.ShapeDtypeStruct((_R, _N, _DW), jnp.int32),
)

_COLS_LO = [32 * (t // 16) + (t % 16) for t in range(_DW)]
_COLS_HI = [c + 16 for c in _COLS_LO]


_sc_mesh = plsc.VectorSubcoreMesh(core_axis_name="c", subcore_axis_name="s")

_sc_params = pltpu.CompilerParams(
    needs_layout_passes=False, use_tc_tiling_on_sc=False)


@functools.partial(
    pl.kernel,
    out_type=jax.ShapeDtypeStruct((_NC, _N, _D), jnp.float32),
    mesh=_sc_mesh,
    compiler_params=_sc_params,
    scratch_types=[
        pltpu.VMEM((_CPH, _CH), jnp.int32),
        pltpu.VMEM((_CPH, _CH), jnp.int32),
        pltpu.VMEM((_CPH, _CH), jnp.float32),
        pltpu.VMEM((_CH, _DW), jnp.int32),
        pltpu.VMEM((_CH, _DW), jnp.int32),
        pltpu.VMEM((_CH, _D), jnp.float32),
        pltpu.SemaphoreType.DMA,
        pltpu.SemaphoreType.DMA,
        pltpu.VMEM_SHARED((_N, _D), jnp.float32),
    ],
)
def _sc_scatter(table_hbm, gidx_hbm, dst_hbm, norm_hbm, out_hbm,
                gidx_v, dst_v, norm_v, rows_a, rows_b, rows_f, sem_a, sem_b,
                acc_sh):
    c = lax.axis_index("c")
    s = lax.axis_index("s")
    wid = c * _NS + s

    @pl.loop(0, _CH)
    def _zrow(e):
        for j in range(_D // 16):
            rows_f[e, pl.ds(j * 16, 16)] = jnp.zeros((16,), jnp.float32)

    for k in range(_RPT // _CH):
        pltpu.sync_copy(rows_f, acc_sh.at[pl.ds(s * _RPT + k * _CH, _CH)])
    _TAIL = _RPT - (_RPT // _CH) * _CH
    pltpu.sync_copy(rows_f.at[pl.ds(0, _TAIL)],
                    acc_sh.at[pl.ds(s * _RPT + _RPT - _TAIL, _TAIL)])

    @pl.when(s == _NS - 1)
    def _():
        pltpu.sync_copy(rows_f.at[pl.ds(0, _N - _NS * _RPT)],
                        acc_sh.at[pl.ds(_NS * _RPT, _N - _NS * _RPT)])

    plsc.subcore_barrier()

    def _scale(i, rows):
        @pl.loop(0, _CH, step=4)
        def _edge(e0):
            for k in range(4):
                e = e0 + k
                nv = plsc.load_gather(
                    norm_v, [jnp.full((16,), i, jnp.int32),
                             jnp.full((16,), e, jnp.int32)])
                for q in range(_D // 32):
                    w = rows[e, pl.ds(q * 16, 16)]
                    lo = plsc.bitcast(lax.shift_left(w, 16), jnp.float32)
                    hi = plsc.bitcast(
                        jnp.bitwise_and(w, jnp.int32(-65536)), jnp.float32)
                    rows_f[e, pl.ds(q * 32, 16)] = lo * nv
                    rows_f[e, pl.ds(q * 32 + 16, 16)] = hi * nv

    def _gather_start(i, rows, sem):
        pltpu.async_copy(table_hbm.at[gidx_v.at[i]], rows, sem)

    def _gather_wait(rows, sem):
        pltpu.make_async_copy(table_hbm.at[pl.ds(0, _CH)], rows, sem).wait()

    def _process(i, rows):
        _scale(i, rows)
        pltpu.sync_copy(rows_f, acc_sh.at[dst_v.at[i]], add=True)

    for h in range(_CPT // _CPH):
        pltpu.sync_copy(gidx_hbm.at[wid, pl.ds(h * _CPH, _CPH)], gidx_v)
        pltpu.sync_copy(dst_hbm.at[wid, pl.ds(h * _CPH, _CPH)], dst_v)
        pltpu.sync_copy(norm_hbm.at[wid, pl.ds(h * _CPH, _CPH)], norm_v)
        _gather_start(0, rows_a, sem_a)

        @pl.loop(0, _CPH - 2, step=2)
        def _chunk(i):
            _gather_wait(rows_a, sem_a)
            _gather_start(i + 1, rows_b, sem_b)
            _process(i, rows_a)
            _gather_wait(rows_b, sem_b)
            _gather_start(i + 2, rows_a, sem_a)
            _process(i + 1, rows_b)

        _gather_wait(rows_a, sem_a)
        _gather_start(_CPH - 1, rows_b, sem_b)
        _process(_CPH - 2, rows_a)
        _gather_wait(rows_b, sem_b)
        _process(_CPH - 1, rows_b)

    plsc.subcore_barrier()
    pltpu.sync_copy(acc_sh.at[pl.ds(s * _RPT, _RPT)],
                    out_hbm.at[c, pl.ds(s * _RPT, _RPT)])

    @pl.when(s == _NS - 1)
    def _():
        pltpu.sync_copy(acc_sh.at[pl.ds(_NS * _RPT, _N - _NS * _RPT)],
                        out_hbm.at[c, pl.ds(_NS * _RPT, _N - _NS * _RPT)])


def _fin_body(x_ref, lw_ref, bias_ref, agg_ref, out_ref):
    sl = lax.dot_general(
        x_ref[...], lw_ref[...], (((1,), (0,)), ((), ())),
        preferred_element_type=jnp.float32)
    out_ref[...] = jnp.maximum(
        sl + agg_ref[0] + agg_ref[1] + bias_ref[...], 0.0)


_fin_call = pl.pallas_call(
    _fin_body,
    grid=(_N // _BN,),
    in_specs=[
        pl.BlockSpec((_BN, _D), lambda i: (i, 0)),
        pl.BlockSpec((_D, _D), lambda i: (0, 0)),
        pl.BlockSpec((1, _D), lambda i: (0, 0)),
        pl.BlockSpec((_NC, _BN, _D), lambda i: (0, i, 0)),
    ],
    out_specs=pl.BlockSpec((_BN, _D), lambda i: (i, 0)),
    out_shape=jax.ShapeDtypeStruct((_N, _D), jnp.float32),
)


def kernel(x, edge_index, etype, norm, bases, coeff, loop_weight, bias):
    src = edge_index[0]
    dst = edge_index[1]
    gidx = etype.astype(jnp.int32) * _N + src.astype(jnp.int32)
    pad = _EPAD - _E
    gidx_p = jnp.concatenate(
        [gidx, jnp.zeros((pad,), jnp.int32)]).reshape(_NW, _CPT, _CH)
    dst_p = jnp.concatenate(
        [dst.astype(jnp.int32), jnp.zeros((pad,), jnp.int32)]
    ).reshape(_NW, _CPT, _CH)
    norm_p = jnp.concatenate(
        [norm[:, 0].astype(jnp.float32), jnp.zeros((pad,), jnp.float32)]
    ).reshape(_NW, _CPT, _CH)

    bases_lo = bases[:, :, jnp.array(_COLS_LO, dtype=jnp.int32)]
    bases_hi = bases[:, :, jnp.array(_COLS_HI, dtype=jnp.int32)]
    xw = _xw_call(coeff.reshape(_R, 1, _B), bases_lo, bases_hi, x)
    table = xw.reshape(_R * _N, _DW)
    agg = _sc_scatter(table, gidx_p, dst_p, norm_p)
    return _fin_call(x, loop_weight, bias.reshape(1, _D), agg)
